# baseline (speedup 1.0000x reference)
import functools

import jax
import jax.numpy as jnp
from jax import lax
from jax.experimental import pallas as pl
from jax.experimental.pallas import tpu as pltpu
from jax.experimental.pallas import tpu_sc as plsc

B = 16384
WIDE_DIM = 128
F = 26
V = 100000
D = 32
N = B * F

NC = 2
NS = 16
NW = NC * NS
PER_W = N // NW
SUB = 128
CHUNK = 1024
KSUB = CHUNK // SUB
NSTEP = PER_W // CHUNK


def _gather_body(table_hbm, idx_hbm, out_hbm, idx_v, rows_v, sem):
    wid = lax.axis_index("s") * NC + lax.axis_index("c")
    row0 = wid * PER_W

    def step(i, _):
        base = pl.multiple_of(row0 + i * CHUNK, CHUNK)
        pltpu.sync_copy(idx_hbm.at[pl.ds(pl.multiple_of(base // SUB, KSUB), KSUB)], idx_v)
        handles = []
        for j in range(KSUB):
            handles.append(pltpu.async_copy(
                table_hbm.at[idx_v.at[j]],
                rows_v.at[pl.ds(j * SUB, SUB)],
                sem,
            ))
        for h in handles:
            h.wait()
        pltpu.sync_copy(rows_v, out_hbm.at[pl.ds(base, CHUNK)])
        return 0

    lax.fori_loop(0, NSTEP, step, 0)


_gather = functools.partial(
    pl.kernel,
    out_type=jax.ShapeDtypeStruct((N, D), jnp.bfloat16),
    mesh=plsc.VectorSubcoreMesh(core_axis_name="c", subcore_axis_name="s"),
    compiler_params=pltpu.CompilerParams(use_tc_tiling_on_sc=False),
    scratch_types=[
        pltpu.VMEM((KSUB, SUB), jnp.int32),
        pltpu.VMEM((CHUNK, D), jnp.bfloat16),
        pltpu.SemaphoreType.DMA,
    ],
)(_gather_body)


def _mlp_body(xw_ref, e_ref, w1_ref, b1_ref, w2_ref, b2_ref, w3_ref, b3_ref,
              wo_ref, bo_ref, out_ref):
    f32 = jnp.float32
    bf = jnp.bfloat16
    h = jnp.dot(e_ref[...], w1_ref[...], preferred_element_type=f32)
    h = jnp.maximum(h + b1_ref[...], 0.0).astype(bf)
    h = jnp.dot(h, w2_ref[...], preferred_element_type=f32)
    h = jnp.maximum(h + b2_ref[...], 0.0).astype(bf)
    h = jnp.dot(h, w3_ref[...], preferred_element_type=f32)
    h = jnp.maximum(h + b3_ref[...], 0.0)
    wo = wo_ref[...]
    logit = (jnp.dot(xw_ref[...], wo[:WIDE_DIM], preferred_element_type=f32)
             + jnp.dot(h, wo[WIDE_DIM:], preferred_element_type=f32)
             + bo_ref[...])
    out_ref[...] = jax.nn.sigmoid(logit)


BM = 2048


def _mlp(xw, e, w1, b1, w2, b2, w3, b3, wo, bo):
    in_dim = F * D
    full = lambda r, c: pl.BlockSpec((r, c), lambda i: (0, 0))
    return pl.pallas_call(
        _mlp_body,
        grid=(B // BM,),
        in_specs=[
            pl.BlockSpec((BM, WIDE_DIM), lambda i: (i, 0)),
            pl.BlockSpec((BM, in_dim), lambda i: (i, 0)),
            full(in_dim, 512), full(1, 512),
            full(512, 256), full(1, 256),
            full(256, 128), full(1, 128),
            full(128 + WIDE_DIM, 1), full(1, 1),
        ],
        out_specs=pl.BlockSpec((BM, 1), lambda i: (i, 0)),
        out_shape=jax.ShapeDtypeStruct((B, 1), jnp.float32),
    )(xw, e, w1, b1, w2, b2, w3, b3, wo, bo)


def kernel(x_wide, x_deep, tables, wide_W, wide_b, W1, b1, W2, b2, W3, b3,
           Wout, bout):
    del wide_W, wide_b
    flat_idx = (x_deep + jnp.arange(F, dtype=jnp.int32)[None, :] * V)
    flat_idx = flat_idx.reshape(N // SUB, SUB)
    bf = jnp.bfloat16
    table2d = tables.reshape(F * V, D).astype(bf)
    emb = _gather(table2d, flat_idx)
    e = emb.reshape(B, F * D)
    return _mlp(x_wide, e, W1.reshape(F * D, 512).astype(bf),
                b1.reshape(1, 512),
                W2.astype(bf), b2.reshape(1, 256),
                W3.astype(bf), b3.reshape(1, 128),
                Wout, bout.reshape(1, 1))


# speedup vs baseline: 1.0839x; 1.0839x over previous
import functools

import jax
import jax.numpy as jnp
from jax import lax
from jax.experimental import pallas as pl
from jax.experimental.pallas import tpu as pltpu
from jax.experimental.pallas import tpu_sc as plsc

B = 16384
WIDE_DIM = 128
F = 26
V = 100000
D = 32
N = B * F

NC = 2
NS = 16
NW = NC * NS
PER_W = N // NW
SUB = 128
CHUNK = 1024
KSUB = CHUNK // SUB
NSTEP = PER_W // CHUNK


def _gather_body(table_hbm, idx_hbm, out_hbm, idx_v, rows_v, sem):
    wid = lax.axis_index("s") * NC + lax.axis_index("c")
    row0 = wid * PER_W

    def step(i, _):
        base = pl.multiple_of(row0 + i * CHUNK, CHUNK)
        pltpu.sync_copy(idx_hbm.at[pl.ds(pl.multiple_of(base // SUB, KSUB), KSUB)], idx_v)
        handles = []
        for j in range(KSUB):
            handles.append(pltpu.async_copy(
                table_hbm.at[idx_v.at[j]],
                rows_v.at[pl.ds(j * SUB, SUB)],
                sem,
            ))
        for h in handles:
            h.wait()
        pltpu.sync_copy(rows_v, out_hbm.at[pl.ds(base, CHUNK)])
        return 0

    lax.fori_loop(0, NSTEP, step, 0)


_gather = functools.partial(
    pl.kernel,
    out_type=jax.ShapeDtypeStruct((N, D), jnp.float32),
    mesh=plsc.VectorSubcoreMesh(core_axis_name="c", subcore_axis_name="s"),
    compiler_params=pltpu.CompilerParams(use_tc_tiling_on_sc=False),
    scratch_types=[
        pltpu.VMEM((KSUB, SUB), jnp.int32),
        pltpu.VMEM((CHUNK, D), jnp.float32),
        pltpu.SemaphoreType.DMA,
    ],
)(_gather_body)


def _mlp_body(xw_ref, e_ref, w1_ref, b1_ref, w2_ref, b2_ref, w3_ref, b3_ref,
              wo_ref, bo_ref, out_ref):
    f32 = jnp.float32
    bf = jnp.bfloat16
    h = jnp.dot(e_ref[...], w1_ref[...], preferred_element_type=f32)
    h = jnp.maximum(h + b1_ref[...], 0.0).astype(bf)
    h = jnp.dot(h, w2_ref[...], preferred_element_type=f32)
    h = jnp.maximum(h + b2_ref[...], 0.0).astype(bf)
    h = jnp.dot(h, w3_ref[...], preferred_element_type=f32)
    h = jnp.maximum(h + b3_ref[...], 0.0)
    wo = wo_ref[...]
    logit = (jnp.dot(xw_ref[...], wo[:WIDE_DIM], preferred_element_type=f32)
             + jnp.dot(h, wo[WIDE_DIM:], preferred_element_type=f32)
             + bo_ref[...])
    out_ref[...] = jax.nn.sigmoid(logit)


BM = 2048


def _mlp(xw, e, w1, b1, w2, b2, w3, b3, wo, bo):
    in_dim = F * D
    full = lambda r, c: pl.BlockSpec((r, c), lambda i: (0, 0))
    return pl.pallas_call(
        _mlp_body,
        grid=(B // BM,),
        in_specs=[
            pl.BlockSpec((BM, WIDE_DIM), lambda i: (i, 0)),
            pl.BlockSpec((BM, in_dim), lambda i: (i, 0)),
            full(in_dim, 512), full(1, 512),
            full(512, 256), full(1, 256),
            full(256, 128), full(1, 128),
            full(128 + WIDE_DIM, 1), full(1, 1),
        ],
        out_specs=pl.BlockSpec((BM, 1), lambda i: (i, 0)),
        out_shape=jax.ShapeDtypeStruct((B, 1), jnp.float32),
    )(xw, e, w1, b1, w2, b2, w3, b3, wo, bo)


def kernel(x_wide, x_deep, tables, wide_W, wide_b, W1, b1, W2, b2, W3, b3,
           Wout, bout):
    del wide_W, wide_b
    flat_idx = (x_deep + jnp.arange(F, dtype=jnp.int32)[None, :] * V)
    flat_idx = flat_idx.reshape(N // SUB, SUB)
    bf = jnp.bfloat16
    table2d = tables.reshape(F * V, D)
    emb = _gather(table2d, flat_idx)
    e = emb.reshape(B, F * D).astype(bf)
    return _mlp(x_wide, e, W1.reshape(F * D, 512).astype(bf),
                b1.reshape(1, 512),
                W2.astype(bf), b2.reshape(1, 256),
                W3.astype(bf), b3.reshape(1, 128),
                Wout, bout.reshape(1, 1))


# speedup vs baseline: 1.2251x; 1.1303x over previous
import functools

import jax
import jax.numpy as jnp
from jax import lax
from jax.experimental import pallas as pl
from jax.experimental.pallas import tpu as pltpu
from jax.experimental.pallas import tpu_sc as plsc

B = 16384
WIDE_DIM = 128
F = 26
V = 100000
D = 32
N = B * F

NC = 2
NS = 16
NW = NC * NS
PER_W = N // NW
SUB = 128
CHUNK = 1024
KSUB = CHUNK // SUB
NSTEP = PER_W // CHUNK


def _gather_body(table_hbm, idx_hbm, out_hbm, idx_v, rows_v, sem):
    wid = lax.axis_index("s") * NC + lax.axis_index("c")
    row0 = wid * PER_W

    def step(i, _):
        base = pl.multiple_of(row0 + i * CHUNK, CHUNK)
        pltpu.sync_copy(
            idx_hbm.at[pl.ds(pl.multiple_of(base // SUB, KSUB), KSUB)], idx_v)
        handles = []
        for j in range(KSUB):
            handles.append(pltpu.async_copy(
                table_hbm.at[idx_v.at[j]],
                rows_v.at[pl.ds(j * SUB, SUB)],
                sem,
            ))
        for h in handles:
            h.wait()
        pltpu.sync_copy(rows_v, out_hbm.at[pl.ds(base, CHUNK)])
        return 0

    lax.fori_loop(0, NSTEP, step, 0)


_gather = functools.partial(
    pl.kernel,
    out_type=jax.ShapeDtypeStruct((N, D), jnp.float32),
    mesh=plsc.VectorSubcoreMesh(core_axis_name="c", subcore_axis_name="s"),
    compiler_params=pltpu.CompilerParams(use_tc_tiling_on_sc=False),
    scratch_types=[
        pltpu.VMEM((KSUB, SUB), jnp.int32),
        pltpu.VMEM((CHUNK, D), jnp.float32),
        pltpu.SemaphoreType.DMA,
    ],
)(_gather_body)


def _mlp_body(xw_ref, e_ref, w1_ref, b1_ref, w2_ref, b2_ref, w3_ref, b3_ref,
              wo_ref, bo_ref, out_ref):
    f32 = jnp.float32
    bf = jnp.bfloat16
    e = e_ref[...].astype(bf)
    h = jnp.dot(e, w1_ref[...].astype(bf), preferred_element_type=f32)
    h = jnp.maximum(h + b1_ref[...], 0.0).astype(bf)
    h = jnp.dot(h, w2_ref[...].astype(bf), preferred_element_type=f32)
    h = jnp.maximum(h + b2_ref[...], 0.0).astype(bf)
    h = jnp.dot(h, w3_ref[...].astype(bf), preferred_element_type=f32)
    h = jnp.maximum(h + b3_ref[...], 0.0)
    wo = wo_ref[...]
    logit = (jnp.dot(xw_ref[...], wo[:WIDE_DIM], preferred_element_type=f32)
             + jnp.dot(h, wo[WIDE_DIM:], preferred_element_type=f32)
             + bo_ref[...])
    out_ref[...] = jax.nn.sigmoid(logit)


BM = 1024


def _mlp(xw, e, w1, b1, w2, b2, w3, b3, wo, bo):
    in_dim = F * D
    full = lambda r, c: pl.BlockSpec((r, c), lambda i: (0, 0))
    return pl.pallas_call(
        _mlp_body,
        grid=(B // BM,),
        in_specs=[
            pl.BlockSpec((BM, WIDE_DIM), lambda i: (i, 0)),
            pl.BlockSpec((BM, in_dim), lambda i: (i, 0)),
            full(in_dim, 512), full(1, 512),
            full(512, 256), full(1, 256),
            full(256, 128), full(1, 128),
            full(128 + WIDE_DIM, 1), full(1, 1),
        ],
        out_specs=pl.BlockSpec((BM, 1), lambda i: (i, 0)),
        out_shape=jax.ShapeDtypeStruct((B, 1), jnp.float32),
    )(xw, e, w1, b1, w2, b2, w3, b3, wo, bo)


def kernel(x_wide, x_deep, tables, wide_W, wide_b, W1, b1, W2, b2, W3, b3,
           Wout, bout):
    del wide_W, wide_b
    flat_idx = (x_deep + jnp.arange(F, dtype=jnp.int32)[None, :] * V)
    flat_idx = flat_idx.reshape(N // SUB, SUB)
    table2d = tables.reshape(F * V, D)
    emb = _gather(table2d, flat_idx)
    e = emb.reshape(B, F * D)
    return _mlp(x_wide, e, W1.reshape(F * D, 512), b1.reshape(1, 512),
                W2, b2.reshape(1, 256), W3, b3.reshape(1, 128),
                Wout, bout.reshape(1, 1))
